# baseline (speedup 1.0000x reference)
import functools

import jax
import jax.numpy as jnp
from jax import lax
from jax.experimental import pallas as pl
from jax.experimental.pallas import tpu as pltpu
from jax.experimental.pallas import tpu_sc as plsc

N = 10000
K = 16
D = 256
C = 64
H = C // 2
T = 3
EPS = 1e-6

NC = 2
NS = 16
NW = NC * NS
NPW = 320
NPAD = NW * NPW
CHUNK_NODES = 8
CHUNK_IDX = CHUNK_NODES * K
NCHUNK = NPW // CHUNK_NODES
LANES = 16
STAGE_ROWS = 624

RB = 2000
GRID = N // RB


def _pack(res):
    lo = lax.bitcast_convert_type(res[:, :H].astype(jnp.bfloat16), jnp.uint16)
    hi = lax.bitcast_convert_type(res[:, H:].astype(jnp.bfloat16), jnp.uint16)
    packed = lo.astype(jnp.uint32) | (hi.astype(jnp.uint32) << 16)
    return lax.bitcast_convert_type(packed, jnp.int32)


def _init_body(x_ref, w_ref, b_ref, o_ref, op_ref):
    logits = jnp.dot(x_ref[...], w_ref[...], preferred_element_type=jnp.float32)
    logits = logits + b_ref[...]
    m = jnp.max(logits, axis=1, keepdims=True)
    e = jnp.exp(logits - m)
    res = e / jnp.sum(e, axis=1, keepdims=True)
    o_ref[...] = res
    op_ref[...] = _pack(res)


_init = pl.pallas_call(
    _init_body,
    grid=(GRID,),
    in_specs=[
        pl.BlockSpec((RB, D), lambda i: (i, 0)),
        pl.BlockSpec((D, C), lambda i: (0, 0)),
        pl.BlockSpec((1, C), lambda i: (0, 0)),
    ],
    out_specs=[
        pl.BlockSpec((RB, C), lambda i: (i, 0)),
        pl.BlockSpec((RB, H), lambda i: (i, 0)),
    ],
    out_shape=[
        jax.ShapeDtypeStruct((N, C), jnp.float32),
        jax.ShapeDtypeStruct((N, H), jnp.int32),
    ],
)


def _mix_body(al_ref, be_ref, p_ref, s_ref, o_ref, op_ref):
    alpha = al_ref[0]
    beta = be_ref[0]
    p = p_ref[...]
    nm = s_ref[...] * (1.0 / K)
    pc = jnp.clip(p, EPS, 1.0 - EPS)
    nc = jnp.clip(nm, EPS, 1.0 - EPS)
    ml = alpha * jnp.log(pc / (1.0 - pc)) + beta * jnp.log(nc / (1.0 - nc))
    m = jnp.max(ml, axis=1, keepdims=True)
    e = jnp.exp(ml - m)
    res = e / jnp.sum(e, axis=1, keepdims=True)
    o_ref[...] = res
    op_ref[...] = _pack(res)


_mix = pl.pallas_call(
    _mix_body,
    grid=(GRID,),
    in_specs=[
        pl.BlockSpec(memory_space=pltpu.SMEM),
        pl.BlockSpec(memory_space=pltpu.SMEM),
        pl.BlockSpec((RB, C), lambda i: (i, 0)),
        pl.BlockSpec((RB, C), lambda i: (i, 0)),
    ],
    out_specs=[
        pl.BlockSpec((RB, C), lambda i: (i, 0)),
        pl.BlockSpec((RB, H), lambda i: (i, 0)),
    ],
    out_shape=[
        jax.ShapeDtypeStruct((N, C), jnp.float32),
        jax.ShapeDtypeStruct((N, H), jnp.int32),
    ],
)


@functools.lru_cache(maxsize=None)
def _make_gather_sum():
    return pl.kernel(
        _gather_sum_body,
        out_type=jax.ShapeDtypeStruct((NPAD, C), jnp.float32),
        mesh=plsc.VectorSubcoreMesh(
            core_axis_name="c", subcore_axis_name="s", num_cores=NC),
        compiler_params=pltpu.CompilerParams(use_tc_tiling_on_sc=False),
        scratch_types=[
            pltpu.VMEM((NCHUNK, CHUNK_IDX), jnp.int32),
            pltpu.VMEM((CHUNK_IDX, H), jnp.int32),
            pltpu.VMEM((CHUNK_IDX, H), jnp.int32),
            pltpu.VMEM((NPW, C), jnp.float32),
            pltpu.VMEM_SHARED((N, H), jnp.int32),
            pltpu.SemaphoreType.DMA,
            pltpu.SemaphoreType.DMA,
            pltpu.SemaphoreType.DMA,
        ],
    )


HBM_CHUNKS = 16


def _gather_sum_body(probs_hbm, idx_hbm, out_hbm, idx_v, buf0, buf1, out_v,
                     table, sem0, sem1, sem_st):
    cid = lax.axis_index("c")
    sid = lax.axis_index("s")
    wid = sid * NC + cid
    pltpu.async_copy(probs_hbm.at[pl.ds(sid * STAGE_ROWS, STAGE_ROWS)],
                     table.at[pl.ds(sid * STAGE_ROWS, STAGE_ROWS)], sem_st)

    @pl.when(sid == NS - 1)
    def _():
        tail = NS * STAGE_ROWS
        pltpu.sync_copy(probs_hbm.at[pl.ds(tail, N - tail)],
                        table.at[pl.ds(tail, N - tail)])

    pltpu.sync_copy(idx_hbm.at[wid], idx_v)

    def fire(g, buf, sem, src):
        pltpu.async_copy(src.at[idx_v.at[g]], buf, sem)

    def wait(g, buf, sem, src):
        pltpu.make_async_copy(src.at[idx_v.at[g]], buf, sem).wait()

    def compute(buf, node_base):
        for nd in range(CHUNK_NODES):
            r0 = nd * K
            for cg in range(H // LANES):
                sl = pl.ds(cg * LANES, LANES)
                v = buf[r0, sl]
                acc_lo = lax.bitcast_convert_type(v << 16, jnp.float32)
                acc_hi = lax.bitcast_convert_type((v >> 16) << 16, jnp.float32)
                for i in range(1, K):
                    v = buf[r0 + i, sl]
                    acc_lo = acc_lo + lax.bitcast_convert_type(
                        v << 16, jnp.float32)
                    acc_hi = acc_hi + lax.bitcast_convert_type(
                        (v >> 16) << 16, jnp.float32)
                out_v[node_base + nd, pl.ds(cg * LANES, LANES)] = acc_lo
                out_v[node_base + nd, pl.ds(H + cg * LANES, LANES)] = acc_hi

    fire(0, buf0, sem0, probs_hbm)

    def body_hbm(i, carry):
        g = i * 2
        fire(g + 1, buf1, sem1, probs_hbm)
        wait(g, buf0, sem0, probs_hbm)
        compute(buf0, g * CHUNK_NODES)

        @pl.when(g + 2 < HBM_CHUNKS)
        def _():
            fire(g + 2, buf0, sem0, probs_hbm)

        wait(g + 1, buf1, sem1, probs_hbm)
        compute(buf1, (g + 1) * CHUNK_NODES)
        return carry

    lax.fori_loop(0, HBM_CHUNKS // 2, body_hbm, 0)

    pltpu.make_async_copy(
        probs_hbm.at[pl.ds(sid * STAGE_ROWS, STAGE_ROWS)],
        table.at[pl.ds(sid * STAGE_ROWS, STAGE_ROWS)], sem_st).wait()
    plsc.subcore_barrier()

    fire(HBM_CHUNKS, buf0, sem0, table)

    def body_sp(i, carry):
        g = HBM_CHUNKS + i * 2
        fire(g + 1, buf1, sem1, table)
        wait(g, buf0, sem0, table)
        compute(buf0, g * CHUNK_NODES)

        @pl.when(g + 2 < NCHUNK)
        def _():
            fire(g + 2, buf0, sem0, table)

        wait(g + 1, buf1, sem1, table)
        compute(buf1, (g + 1) * CHUNK_NODES)
        return carry

    lax.fori_loop(0, (NCHUNK - HBM_CHUNKS) // 2, body_sp, 0)
    pltpu.sync_copy(out_v, out_hbm.at[pl.ds(wid * NPW, NPW)])


def kernel(X, neighbors, W, b, alpha, beta):
    probs, packed = _init(X, W, jnp.reshape(b, (1, C)))
    flat = jnp.reshape(neighbors, (-1,))
    flat = jnp.concatenate(
        [flat, jnp.zeros(((NPAD - N) * K,), dtype=jnp.int32)])
    idx3 = jnp.reshape(flat, (NW, NCHUNK, CHUNK_IDX))
    a1 = jnp.reshape(alpha, (1,))
    b1 = jnp.reshape(beta, (1,))
    gather_sum = _make_gather_sum()
    for _ in range(T):
        nsum = gather_sum(packed, idx3)
        probs, packed = _mix(a1, b1, probs, nsum)
    return probs


# speedup vs baseline: 1.2090x; 1.2090x over previous
import functools

import jax
import jax.numpy as jnp
from jax import lax
from jax.experimental import pallas as pl
from jax.experimental.pallas import tpu as pltpu
from jax.experimental.pallas import tpu_sc as plsc

N = 10000
K = 16
D = 256
C = 64
H = C // 2
T = 3
EPS = 1e-6

NC = 2
NS = 16
NW = NC * NS
NPW = 320
NPAD = NW * NPW
CHUNK_NODES = 8
CHUNK_IDX = CHUNK_NODES * K
NCHUNK = NPW // CHUNK_NODES
LANES = 16
STAGE_ROWS = 624

RB = 2000
GRID = N // RB


def _pack(res):
    lo = lax.bitcast_convert_type(res[:, :H].astype(jnp.bfloat16), jnp.uint16)
    hi = lax.bitcast_convert_type(res[:, H:].astype(jnp.bfloat16), jnp.uint16)
    packed = lo.astype(jnp.uint32) | (hi.astype(jnp.uint32) << 16)
    return lax.bitcast_convert_type(packed, jnp.int32)


def _init_body(x_ref, w_ref, b_ref, o_ref, op_ref):
    logits = jnp.dot(x_ref[...], w_ref[...], preferred_element_type=jnp.float32)
    logits = logits + b_ref[...]
    m = jnp.max(logits, axis=1, keepdims=True)
    e = jnp.exp(logits - m)
    res = e / jnp.sum(e, axis=1, keepdims=True)
    o_ref[...] = res
    op_ref[...] = _pack(res)


_init = pl.pallas_call(
    _init_body,
    grid=(GRID,),
    in_specs=[
        pl.BlockSpec((RB, D), lambda i: (i, 0)),
        pl.BlockSpec((D, C), lambda i: (0, 0)),
        pl.BlockSpec((1, C), lambda i: (0, 0)),
    ],
    out_specs=[
        pl.BlockSpec((RB, C), lambda i: (i, 0)),
        pl.BlockSpec((RB, H), lambda i: (i, 0)),
    ],
    out_shape=[
        jax.ShapeDtypeStruct((N, C), jnp.float32),
        jax.ShapeDtypeStruct((N, H), jnp.int32),
    ],
)


def _mix_body(al_ref, be_ref, p_ref, s_ref, o_ref, op_ref):
    alpha = al_ref[0]
    beta = be_ref[0]
    p = p_ref[...]
    nm = s_ref[...] * (1.0 / K)
    pc = jnp.clip(p, EPS, 1.0 - EPS)
    nc = jnp.clip(nm, EPS, 1.0 - EPS)
    ml = alpha * jnp.log(pc / (1.0 - pc)) + beta * jnp.log(nc / (1.0 - nc))
    m = jnp.max(ml, axis=1, keepdims=True)
    e = jnp.exp(ml - m)
    res = e / jnp.sum(e, axis=1, keepdims=True)
    o_ref[...] = res
    op_ref[...] = _pack(res)


_mix = pl.pallas_call(
    _mix_body,
    grid=(GRID,),
    in_specs=[
        pl.BlockSpec(memory_space=pltpu.SMEM),
        pl.BlockSpec(memory_space=pltpu.SMEM),
        pl.BlockSpec((RB, C), lambda i: (i, 0)),
        pl.BlockSpec((RB, C), lambda i: (i, 0)),
    ],
    out_specs=[
        pl.BlockSpec((RB, C), lambda i: (i, 0)),
        pl.BlockSpec((RB, H), lambda i: (i, 0)),
    ],
    out_shape=[
        jax.ShapeDtypeStruct((N, C), jnp.float32),
        jax.ShapeDtypeStruct((N, H), jnp.int32),
    ],
)


@functools.lru_cache(maxsize=None)
def _make_gather_sum():
    return pl.kernel(
        _gather_sum_body,
        out_type=jax.ShapeDtypeStruct((NPAD, C), jnp.float32),
        mesh=plsc.VectorSubcoreMesh(
            core_axis_name="c", subcore_axis_name="s", num_cores=NC),
        compiler_params=pltpu.CompilerParams(use_tc_tiling_on_sc=False),
        scratch_types=[
            pltpu.VMEM((NCHUNK, CHUNK_IDX), jnp.int32),
            pltpu.VMEM((CHUNK_IDX, H), jnp.int32),
            pltpu.VMEM((CHUNK_IDX, H), jnp.int32),
            pltpu.VMEM((NPW, C), jnp.float32),
            pltpu.VMEM_SHARED((N, H), jnp.int32),
            pltpu.SemaphoreType.DMA,
            pltpu.SemaphoreType.DMA,
            pltpu.SemaphoreType.DMA,
        ],
    )


def _gather_sum_body(probs_hbm, idx_hbm, out_hbm, idx_v, buf0, buf1, out_v,
                     table, sem0, sem1, sem_st):
    cid = lax.axis_index("c")
    sid = lax.axis_index("s")
    wid = sid * NC + cid
    pltpu.async_copy(probs_hbm.at[pl.ds(sid * STAGE_ROWS, STAGE_ROWS)],
                     table.at[pl.ds(sid * STAGE_ROWS, STAGE_ROWS)], sem_st)

    @pl.when(sid == NS - 1)
    def _():
        tail = NS * STAGE_ROWS
        pltpu.sync_copy(probs_hbm.at[pl.ds(tail, N - tail)],
                        table.at[pl.ds(tail, N - tail)])

    pltpu.sync_copy(idx_hbm.at[wid], idx_v)

    def fire(g, buf, sem, src):
        pltpu.async_copy(src.at[idx_v.at[g]], buf, sem)

    def wait(g, buf, sem, src):
        pltpu.make_async_copy(src.at[idx_v.at[g]], buf, sem).wait()

    def compute(buf, node_base):
        for nd in range(CHUNK_NODES):
            r0 = nd * K
            for cg in range(H // LANES):
                sl = pl.ds(cg * LANES, LANES)
                v = buf[r0, sl]
                acc_lo = lax.bitcast_convert_type(v << 16, jnp.float32)
                acc_hi = lax.bitcast_convert_type((v >> 16) << 16, jnp.float32)
                for i in range(1, K):
                    v = buf[r0 + i, sl]
                    acc_lo = acc_lo + lax.bitcast_convert_type(
                        v << 16, jnp.float32)
                    acc_hi = acc_hi + lax.bitcast_convert_type(
                        (v >> 16) << 16, jnp.float32)
                out_v[node_base + nd, pl.ds(cg * LANES, LANES)] = acc_lo
                out_v[node_base + nd, pl.ds(H + cg * LANES, LANES)] = acc_hi

    pltpu.make_async_copy(
        probs_hbm.at[pl.ds(sid * STAGE_ROWS, STAGE_ROWS)],
        table.at[pl.ds(sid * STAGE_ROWS, STAGE_ROWS)], sem_st).wait()
    plsc.subcore_barrier()

    fire(0, buf0, sem0, table)

    def body(i, carry):
        g = i * 2
        fire(g + 1, buf1, sem1, table)
        wait(g, buf0, sem0, table)
        compute(buf0, g * CHUNK_NODES)

        @pl.when(g + 2 < NCHUNK)
        def _():
            fire(g + 2, buf0, sem0, table)

        wait(g + 1, buf1, sem1, table)
        compute(buf1, (g + 1) * CHUNK_NODES)
        return carry

    lax.fori_loop(0, NCHUNK // 2, body, 0)
    pltpu.sync_copy(out_v, out_hbm.at[pl.ds(wid * NPW, NPW)])


def kernel(X, neighbors, W, b, alpha, beta):
    probs, packed = _init(X, W, jnp.reshape(b, (1, C)))
    flat = jnp.reshape(neighbors, (-1,))
    flat = jnp.concatenate(
        [flat, jnp.zeros(((NPAD - N) * K,), dtype=jnp.int32)])
    idx3 = jnp.reshape(flat, (NW, NCHUNK, CHUNK_IDX))
    a1 = jnp.reshape(alpha, (1,))
    b1 = jnp.reshape(beta, (1,))
    gather_sum = _make_gather_sum()
    for _ in range(T):
        nsum = gather_sum(packed, idx3)
        probs, packed = _mix(a1, b1, probs, nsum)
    return probs
